# baseline (speedup 1.0000x reference)
import jax
import jax.numpy as jnp
from jax.experimental import pallas as pl
from jax.experimental.pallas import tpu as pltpu

N = 50000
D_IN = 256
D_H = 512
D_OUT = 1024
NSEG = 512
BLK = 400
NBLK = N // BLK
CHUNK = 144


def _body(lo_ref, hi_ref, gs_ref, x_ref, seg_ref, w1_ref, b1_ref, w2_ref,
          b2_ref, out_ref, zbuf_ref):
    i = pl.program_id(0)
    rowstart = i * BLK

    @pl.when(i == 0)
    def _init():
        out_ref[:, :] = jnp.full((NSEG, D_OUT), -jnp.inf, jnp.float32)

    z1 = jnp.dot(x_ref[:, :], w1_ref[:, :], preferred_element_type=jnp.float32)
    z1 = z1 + b1_ref[:, :]
    h = jnp.maximum(z1, 0.01 * z1)
    z2 = jnp.dot(h, w2_ref[:, :], preferred_element_type=jnp.float32)
    zbuf_ref[pl.ds(0, BLK), :] = z2

    lo = lo_ref[i]
    hi = hi_ref[i]

    def win_max(s):
        a = jnp.maximum(gs_ref[s], rowstart) - rowstart
        b = jnp.minimum(gs_ref[s + 1], rowstart + BLK) - rowstart
        a8 = (a // 8) * 8
        zs = zbuf_ref[pl.ds(a8, CHUNK), :]
        row = a8 + jax.lax.broadcasted_iota(jnp.int32, (CHUNK, 1), 0)
        m = jnp.max(jnp.where((row >= a) & (row < b), zs, -jnp.inf),
                    axis=0, keepdims=True)
        return m, (b - a8 > CHUNK).astype(jnp.int32)

    m0, ovf0 = win_max(lo)
    out_ref[pl.ds(lo, 1), :] = jnp.maximum(out_ref[pl.ds(lo, 1), :], m0)

    def seg_body(s, ovf):
        m, of = win_max(s)
        out_ref[pl.ds(s, 1), :] = m
        return ovf | of

    ovf = jax.lax.fori_loop(lo + 1, hi + 1, seg_body, ovf0)

    @pl.when(ovf > 0)
    def _long_segments():
        seg = seg_ref[:, :]
        zfull = zbuf_ref[pl.ds(0, BLK), :]

        def fb(s, carry):
            m = jnp.max(jnp.where(seg == s, zfull, -jnp.inf),
                        axis=0, keepdims=True)
            out_ref[pl.ds(s, 1), :] = jnp.maximum(out_ref[pl.ds(s, 1), :], m)
            return carry

        jax.lax.fori_loop(lo, hi + 1, fb, 0)

    @pl.when(i == NBLK - 1)
    def _final():
        v = out_ref[:, :] + b2_ref[:, :]
        out_ref[:, :] = jnp.maximum(v, 0.01 * v)


def _pooled(x, seg, W1, b1, W2, b2):
    lo = seg[::BLK]
    hi = seg[BLK - 1 :: BLK]
    gstart = jnp.searchsorted(seg, jnp.arange(NSEG + 1, dtype=jnp.int32)).astype(
        jnp.int32
    )
    return pl.pallas_call(
        _body,
        grid=(NBLK,),
        in_specs=[
            pl.BlockSpec(memory_space=pltpu.SMEM),
            pl.BlockSpec(memory_space=pltpu.SMEM),
            pl.BlockSpec(memory_space=pltpu.SMEM),
            pl.BlockSpec((BLK, D_IN), lambda i: (i, 0)),
            pl.BlockSpec((BLK, 1), lambda i: (i, 0)),
            pl.BlockSpec((D_IN, D_H), lambda i: (0, 0)),
            pl.BlockSpec((1, D_H), lambda i: (0, 0)),
            pl.BlockSpec((D_H, D_OUT), lambda i: (0, 0)),
            pl.BlockSpec((1, D_OUT), lambda i: (0, 0)),
        ],
        out_specs=pl.BlockSpec((NSEG, D_OUT), lambda i: (0, 0)),
        out_shape=jax.ShapeDtypeStruct((NSEG, D_OUT), jnp.float32),
        scratch_shapes=[pltpu.VMEM((BLK + CHUNK, D_OUT), jnp.float32)],
        compiler_params=pltpu.CompilerParams(
            dimension_semantics=("arbitrary",),
        ),
    )(
        lo,
        hi,
        gstart,
        x,
        seg.reshape(N, 1),
        W1,
        b1.reshape(1, D_H),
        W2,
        b2.reshape(1, D_OUT),
    )


def kernel(x, pos, batch, W1, b1, W2, b2):
    seg = jnp.asarray(batch, jnp.int32)
    pooled = _pooled(x, seg, W1, b1, W2, b2)
    pos_out = jnp.zeros((NSEG, 3), dtype=pos.dtype)
    batch_out = jnp.arange(NSEG, dtype=batch.dtype)
    return (pooled, pos_out, batch_out)


# speedup vs baseline: 1.1692x; 1.1692x over previous
import jax
import jax.numpy as jnp
from jax.experimental import pallas as pl
from jax.experimental.pallas import tpu as pltpu

N = 50000
D_IN = 256
D_H = 512
D_OUT = 1024
NSEG = 512
BLK = 2000
NBLK = N // BLK
CHUNK = 144


def _body(lo_ref, hi_ref, gs_ref, x_ref, seg_ref, w1_ref, b1_ref, w2_ref,
          b2_ref, out_ref, zbuf_ref):
    i = pl.program_id(0)
    rowstart = i * BLK

    @pl.when(i == 0)
    def _init():
        out_ref[:, :] = jnp.full((NSEG, D_OUT), -jnp.inf, jnp.float32)

    z1 = jnp.dot(x_ref[:, :], w1_ref[:, :], preferred_element_type=jnp.float32)
    z1 = z1 + b1_ref[:, :]
    h = jnp.maximum(z1, 0.01 * z1)
    z2 = jnp.dot(h, w2_ref[:, :], preferred_element_type=jnp.float32)
    zbuf_ref[pl.ds(0, BLK), :] = z2

    lo = lo_ref[i]
    hi = hi_ref[i]

    def win_max(s):
        a = jnp.maximum(gs_ref[s], rowstart) - rowstart
        b = jnp.minimum(gs_ref[s + 1], rowstart + BLK) - rowstart
        a8 = (a // 8) * 8
        zs = zbuf_ref[pl.ds(a8, CHUNK), :]
        row = a8 + jax.lax.broadcasted_iota(jnp.int32, (CHUNK, 1), 0)
        m = jnp.max(jnp.where((row >= a) & (row < b), zs, -jnp.inf),
                    axis=0, keepdims=True)
        return m, (b - a8 > CHUNK).astype(jnp.int32)

    m0, ovf0 = win_max(lo)
    out_ref[pl.ds(lo, 1), :] = jnp.maximum(out_ref[pl.ds(lo, 1), :], m0)

    def seg_body(s, ovf):
        m, of = win_max(s)
        out_ref[pl.ds(s, 1), :] = m
        return ovf | of

    ovf = jax.lax.fori_loop(lo + 1, hi + 1, seg_body, ovf0)

    @pl.when(ovf > 0)
    def _long_segments():
        seg = seg_ref[:, :]
        zfull = zbuf_ref[pl.ds(0, BLK), :]

        def fb(s, carry):
            m = jnp.max(jnp.where(seg == s, zfull, -jnp.inf),
                        axis=0, keepdims=True)
            out_ref[pl.ds(s, 1), :] = jnp.maximum(out_ref[pl.ds(s, 1), :], m)
            return carry

        jax.lax.fori_loop(lo, hi + 1, fb, 0)

    @pl.when(i == NBLK - 1)
    def _final():
        v = out_ref[:, :] + b2_ref[:, :]
        out_ref[:, :] = jnp.maximum(v, 0.01 * v)


def _pooled(x, seg, W1, b1, W2, b2):
    lo = seg[::BLK]
    hi = seg[BLK - 1 :: BLK]
    gstart = jnp.searchsorted(seg, jnp.arange(NSEG + 1, dtype=jnp.int32)).astype(
        jnp.int32
    )
    return pl.pallas_call(
        _body,
        grid=(NBLK,),
        in_specs=[
            pl.BlockSpec(memory_space=pltpu.SMEM),
            pl.BlockSpec(memory_space=pltpu.SMEM),
            pl.BlockSpec(memory_space=pltpu.SMEM),
            pl.BlockSpec((BLK, D_IN), lambda i: (i, 0)),
            pl.BlockSpec((BLK, 1), lambda i: (i, 0)),
            pl.BlockSpec((D_IN, D_H), lambda i: (0, 0)),
            pl.BlockSpec((1, D_H), lambda i: (0, 0)),
            pl.BlockSpec((D_H, D_OUT), lambda i: (0, 0)),
            pl.BlockSpec((1, D_OUT), lambda i: (0, 0)),
        ],
        out_specs=pl.BlockSpec((NSEG, D_OUT), lambda i: (0, 0)),
        out_shape=jax.ShapeDtypeStruct((NSEG, D_OUT), jnp.float32),
        scratch_shapes=[pltpu.VMEM((BLK + CHUNK, D_OUT), jnp.float32)],
        compiler_params=pltpu.CompilerParams(
            dimension_semantics=("arbitrary",),
        ),
    )(
        lo,
        hi,
        gstart,
        x,
        seg.reshape(N, 1),
        W1,
        b1.reshape(1, D_H),
        W2,
        b2.reshape(1, D_OUT),
    )


def kernel(x, pos, batch, W1, b1, W2, b2):
    seg = jnp.asarray(batch, jnp.int32)
    pooled = _pooled(x, seg, W1, b1, W2, b2)
    pos_out = jnp.zeros((NSEG, 3), dtype=pos.dtype)
    batch_out = jnp.arange(NSEG, dtype=batch.dtype)
    return (pooled, pos_out, batch_out)


# speedup vs baseline: 1.1898x; 1.0177x over previous
import jax
import jax.numpy as jnp
from jax.experimental import pallas as pl
from jax.experimental.pallas import tpu as pltpu

N = 50000
D_IN = 256
D_H = 512
D_OUT = 1024
NSEG = 512
BLK = 2000
NBLK = N // BLK
CHUNK = 144


def _body(lo_ref, hi_ref, gs_ref, x_ref, seg_ref, w1_ref, b1_ref, w2_ref,
          b2_ref, out_ref, zbuf_ref):
    i = pl.program_id(0)
    rowstart = i * BLK

    @pl.when(i == 0)
    def _init():
        out_ref[:, :] = jnp.full((NSEG, D_OUT), -jnp.inf, jnp.float32)

    z1 = jnp.dot(x_ref[:, :], w1_ref[:, :], preferred_element_type=jnp.float32)
    z1 = z1 + b1_ref[:, :]
    h = jnp.maximum(z1, 0.01 * z1)
    z2 = jnp.dot(h, w2_ref[:, :], preferred_element_type=jnp.float32)
    zbuf_ref[pl.ds(0, BLK), :] = z2

    lo = lo_ref[i]
    hi = hi_ref[i]

    def win_max(s):
        a = jnp.maximum(gs_ref[s], rowstart) - rowstart
        b = jnp.minimum(gs_ref[s + 1], rowstart + BLK) - rowstart
        a8 = (a // 8) * 8
        zs = zbuf_ref[pl.ds(a8, CHUNK), :]
        row = a8 + jax.lax.broadcasted_iota(jnp.int32, (CHUNK, 1), 0)
        m = jnp.max(jnp.where((row >= a) & (row < b), zs, -jnp.inf),
                    axis=0, keepdims=True)
        return m, (b - a8 > CHUNK).astype(jnp.int32)

    m0, ovf0 = win_max(lo)
    out_ref[pl.ds(lo, 1), :] = jnp.maximum(out_ref[pl.ds(lo, 1), :], m0)

    def seg_body(t, ovf):
        s1 = lo + 1 + 2 * t
        m1, of1 = win_max(s1)
        out_ref[pl.ds(s1, 1), :] = m1
        s2 = jnp.minimum(s1 + 1, hi)
        m2, of2 = win_max(s2)
        out_ref[pl.ds(s2, 1), :] = m2
        return ovf | of1 | of2

    ovf = jax.lax.fori_loop(0, (hi - lo + 1) // 2, seg_body, ovf0)

    @pl.when(ovf > 0)
    def _long_segments():
        seg = seg_ref[:, :]
        zfull = zbuf_ref[pl.ds(0, BLK), :]

        def fb(s, carry):
            m = jnp.max(jnp.where(seg == s, zfull, -jnp.inf),
                        axis=0, keepdims=True)
            out_ref[pl.ds(s, 1), :] = jnp.maximum(out_ref[pl.ds(s, 1), :], m)
            return carry

        jax.lax.fori_loop(lo, hi + 1, fb, 0)

    @pl.when(i == NBLK - 1)
    def _final():
        v = out_ref[:, :] + b2_ref[:, :]
        out_ref[:, :] = jnp.maximum(v, 0.01 * v)


def _pooled(x, seg, W1, b1, W2, b2):
    lo = seg[::BLK]
    hi = seg[BLK - 1 :: BLK]
    gstart = jnp.searchsorted(seg, jnp.arange(NSEG + 1, dtype=jnp.int32)).astype(
        jnp.int32
    )
    return pl.pallas_call(
        _body,
        grid=(NBLK,),
        in_specs=[
            pl.BlockSpec(memory_space=pltpu.SMEM),
            pl.BlockSpec(memory_space=pltpu.SMEM),
            pl.BlockSpec(memory_space=pltpu.SMEM),
            pl.BlockSpec((BLK, D_IN), lambda i: (i, 0)),
            pl.BlockSpec((BLK, 1), lambda i: (i, 0)),
            pl.BlockSpec((D_IN, D_H), lambda i: (0, 0)),
            pl.BlockSpec((1, D_H), lambda i: (0, 0)),
            pl.BlockSpec((D_H, D_OUT), lambda i: (0, 0)),
            pl.BlockSpec((1, D_OUT), lambda i: (0, 0)),
        ],
        out_specs=pl.BlockSpec((NSEG, D_OUT), lambda i: (0, 0)),
        out_shape=jax.ShapeDtypeStruct((NSEG, D_OUT), jnp.float32),
        scratch_shapes=[pltpu.VMEM((BLK + CHUNK, D_OUT), jnp.float32)],
        compiler_params=pltpu.CompilerParams(
            dimension_semantics=("arbitrary",),
        ),
    )(
        lo,
        hi,
        gstart,
        x,
        seg.reshape(N, 1),
        W1,
        b1.reshape(1, D_H),
        W2,
        b2.reshape(1, D_OUT),
    )


def kernel(x, pos, batch, W1, b1, W2, b2):
    seg = jnp.asarray(batch, jnp.int32)
    pooled = _pooled(x, seg, W1, b1, W2, b2)
    pos_out = jnp.zeros((NSEG, 3), dtype=pos.dtype)
    batch_out = jnp.arange(NSEG, dtype=batch.dtype)
    return (pooled, pos_out, batch_out)
